# baseline (speedup 1.0000x reference)
import functools

import jax
import jax.numpy as jnp
from jax import lax
from jax.experimental import pallas as pl
from jax.experimental.pallas import tpu as pltpu, tpu_sc as plsc

N_TOKENS = 8192
IN_DIM = 2048
HIDDEN_DIM = 2048
POOL = 64
GROUPS = 6
OUT_DIM = POOL * GROUPS
TOPK = 8

M_BLK = 512


def _mlp_body(x_ref, w1_ref, b1_ref, w2_ref, b2_ref, w3_ref, b3_ref, coeff_ref):
    x = x_ref[...]
    h1 = jax.nn.gelu(
        jnp.dot(x, w1_ref[...], preferred_element_type=jnp.float32, precision="default") + b1_ref[...],
        approximate=True,
    )
    h2 = jax.nn.gelu(
        jnp.dot(h1, w2_ref[...], preferred_element_type=jnp.float32, precision="default") + b2_ref[...],
        approximate=True,
    )
    z = jnp.dot(h2, w3_ref[...], preferred_element_type=jnp.float32, precision="default") + b3_ref[...]
    coeff_ref[...] = 2.0 * jax.nn.sigmoid(z)


def _mlp(x, W1, b1, W2, b2, W3, b3):
    n_tok = x.shape[0]
    grid = (n_tok // M_BLK,)
    return pl.pallas_call(
        _mlp_body,
        grid=grid,
        in_specs=[
            pl.BlockSpec((M_BLK, IN_DIM), lambda i: (i, 0)),
            pl.BlockSpec((IN_DIM, HIDDEN_DIM), lambda i: (0, 0)),
            pl.BlockSpec((1, HIDDEN_DIM), lambda i: (0, 0)),
            pl.BlockSpec((HIDDEN_DIM, HIDDEN_DIM), lambda i: (0, 0)),
            pl.BlockSpec((1, HIDDEN_DIM), lambda i: (0, 0)),
            pl.BlockSpec((HIDDEN_DIM, OUT_DIM), lambda i: (0, 0)),
            pl.BlockSpec((1, OUT_DIM), lambda i: (0, 0)),
        ],
        out_specs=pl.BlockSpec((M_BLK, OUT_DIM), lambda i: (i, 0)),
        out_shape=jax.ShapeDtypeStruct((n_tok, OUT_DIM), jnp.float32),
    )(x, W1, b1.reshape(1, -1), W2, b2.reshape(1, -1), W3, b3.reshape(1, -1))


ROWS = N_TOKENS * GROUPS
NC, NS, L = 2, 16, 16
NW = NC * NS
R_TILE = 128
N_CHUNKS = 4


def _make_topk_body(rpw, n_tiles):
    def _topk_body(coeff_hbm, vals_hbm, idx_hbm, in_v, outv_v, outi_v):
        w = lax.axis_index("s") * NC + lax.axis_index("c")
        row0 = w * rpw
        iota = lax.iota(jnp.int32, L)
        lo8 = iota < 8
        idx_c = [iota + 16 * c for c in range(4)]

        def merge(ak, ai, bk, bi):
            mk = jnp.where(lo8, ak, lax.rev(bk, (0,)))
            mi = jnp.where(lo8, ai, lax.rev(bi, (0,)))
            return plsc.sort_key_val(mk, mi, descending=True)

        def tile_body(t, _):
            pltpu.sync_copy(
                coeff_hbm.at[pl.ds((row0 + t * R_TILE) * POOL, R_TILE * POOL)],
                in_v,
            )

            def row_body(r, _):
                base = r * POOL
                s = []
                for c in range(4):
                    v = in_v[pl.ds(base + 16 * c, L)]
                    s.append(plsc.sort_key_val(v, idx_c[c], descending=True))
                ak, ai = merge(*s[0], *s[1])
                bk, bi = merge(*s[2], *s[3])
                fk, fi = merge(ak, ai, bk, bi)
                off = (t * R_TILE + r) * TOPK
                outv_v[pl.ds(off, L)] = fk
                outi_v[pl.ds(off, L)] = fi
                return 0

            lax.fori_loop(0, R_TILE, row_body, 0, unroll=8)
            return 0

        lax.fori_loop(0, n_tiles, tile_body, 0)
        pltpu.sync_copy(
            outv_v.at[pl.ds(0, rpw * TOPK)],
            vals_hbm.at[pl.ds(row0 * TOPK, rpw * TOPK)],
        )
        pltpu.sync_copy(
            outi_v.at[pl.ds(0, rpw * TOPK)],
            idx_hbm.at[pl.ds(row0 * TOPK, rpw * TOPK)],
        )

    return _topk_body


def _topk(coeff_flat):
    rows = coeff_flat.shape[0] // POOL
    rpw = rows // NW
    n_tiles = rpw // R_TILE
    mesh = plsc.VectorSubcoreMesh(
        core_axis_name="c", subcore_axis_name="s", num_cores=NC, num_subcores=NS
    )
    return pl.kernel(
        _make_topk_body(rpw, n_tiles),
        out_type=(
            jax.ShapeDtypeStruct((rows * TOPK,), jnp.float32),
            jax.ShapeDtypeStruct((rows * TOPK,), jnp.int32),
        ),
        mesh=mesh,
        compiler_params=pltpu.CompilerParams(needs_layout_passes=False),
        scratch_types=[
            pltpu.VMEM((R_TILE * POOL,), jnp.float32),
            pltpu.VMEM((rpw * TOPK + L,), jnp.float32),
            pltpu.VMEM((rpw * TOPK + L,), jnp.int32),
        ],
    )(coeff_flat)


def kernel(query_in, W1, b1, W2, b2, W3, b3):
    per = N_TOKENS // N_CHUNKS
    coeffs, tvs, tis = [], [], []
    for i in range(N_CHUNKS):
        c = _mlp(
            lax.slice_in_dim(query_in, i * per, (i + 1) * per),
            W1, b1, W2, b2, W3, b3,
        )
        v, ix = _topk(c.reshape(-1))
        coeffs.append(c)
        tvs.append(v.reshape(per, GROUPS * TOPK))
        tis.append(ix.reshape(per, GROUPS * TOPK))
    coeff = jnp.concatenate(coeffs, axis=0)
    topk_coeff = jnp.concatenate(tvs, axis=0)
    topk_idx = jnp.concatenate(tis, axis=0)
    return coeff, topk_coeff, topk_idx


# speedup vs baseline: 1.1855x; 1.1855x over previous
import functools

import jax
import jax.numpy as jnp
from jax import lax
from jax.experimental import pallas as pl
from jax.experimental.pallas import tpu as pltpu, tpu_sc as plsc

N_TOKENS = 8192
IN_DIM = 2048
HIDDEN_DIM = 2048
POOL = 64
GROUPS = 6
OUT_DIM = POOL * GROUPS
TOPK = 8

M_BLK = 512


def _mlp_body(x_ref, w1_ref, b1_ref, w2_ref, b2_ref, w3_ref, b3_ref, coeff_ref):
    x = x_ref[...]
    h1 = jax.nn.gelu(
        jnp.dot(x, w1_ref[...], preferred_element_type=jnp.float32, precision="default") + b1_ref[...],
        approximate=True,
    )
    h2 = jax.nn.gelu(
        jnp.dot(h1, w2_ref[...], preferred_element_type=jnp.float32, precision="default") + b2_ref[...],
        approximate=True,
    )
    z = jnp.dot(h2, w3_ref[...], preferred_element_type=jnp.float32, precision="default") + b3_ref[...]
    coeff_ref[...] = 2.0 * jax.nn.sigmoid(z)


def _mlp(x, W1, b1, W2, b2, W3, b3):
    n_tok = x.shape[0]
    grid = (n_tok // M_BLK,)
    return pl.pallas_call(
        _mlp_body,
        grid=grid,
        in_specs=[
            pl.BlockSpec((M_BLK, IN_DIM), lambda i: (i, 0)),
            pl.BlockSpec((IN_DIM, HIDDEN_DIM), lambda i: (0, 0)),
            pl.BlockSpec((1, HIDDEN_DIM), lambda i: (0, 0)),
            pl.BlockSpec((HIDDEN_DIM, HIDDEN_DIM), lambda i: (0, 0)),
            pl.BlockSpec((1, HIDDEN_DIM), lambda i: (0, 0)),
            pl.BlockSpec((HIDDEN_DIM, OUT_DIM), lambda i: (0, 0)),
            pl.BlockSpec((1, OUT_DIM), lambda i: (0, 0)),
        ],
        out_specs=pl.BlockSpec((M_BLK, OUT_DIM), lambda i: (i, 0)),
        out_shape=jax.ShapeDtypeStruct((n_tok, OUT_DIM), jnp.float32),
    )(x, W1, b1.reshape(1, -1), W2, b2.reshape(1, -1), W3, b3.reshape(1, -1))


ROWS = N_TOKENS * GROUPS
NC, NS, L = 2, 16, 16
NW = NC * NS
RPW = ROWS // NW
R_TILE = 256
N_TILES = RPW // R_TILE


def _topk_body(coeff_hbm, vals_hbm, idx_hbm, in_a, in_b, outv_v, outi_v, sem_a, sem_b):
    w = lax.axis_index("s") * NC + lax.axis_index("c")
    row0 = w * RPW
    iota = lax.iota(jnp.int32, L)
    lo8 = iota < 8
    idx_c = [iota + 16 * c for c in range(4)]

    def in_copy(t, buf, sem):
        return pltpu.make_async_copy(
            coeff_hbm.at[pl.ds((row0 + t * R_TILE) * POOL, R_TILE * POOL)],
            buf,
            sem,
        )

    def merge(ak, ai, bk, bi):
        mk = jnp.where(lo8, ak, lax.rev(bk, (0,)))
        mi = jnp.where(lo8, ai, lax.rev(bi, (0,)))
        return plsc.sort_key_val(mk, mi, descending=True)

    def process(t, buf):
        def row_body(r, _):
            base = r * POOL
            s = []
            for c in range(4):
                v = buf[pl.ds(base + 16 * c, L)]
                s.append(plsc.sort_key_val(v, idx_c[c], descending=True))
            ak, ai = merge(*s[0], *s[1])
            bk, bi = merge(*s[2], *s[3])
            fk, fi = merge(ak, ai, bk, bi)
            off = (t * R_TILE + r) * TOPK
            outv_v[pl.ds(off, L)] = fk
            outi_v[pl.ds(off, L)] = fi
            return 0

        lax.fori_loop(0, R_TILE, row_body, 0, unroll=8)

    in_copy(0, in_a, sem_a).start()

    def pair_body(tt, _):
        t = 2 * tt
        in_copy(t + 1, in_b, sem_b).start()
        in_copy(t, in_a, sem_a).wait()
        process(t, in_a)

        @pl.when(t + 2 < N_TILES)
        def _():
            in_copy(t + 2, in_a, sem_a).start()

        in_copy(t + 1, in_b, sem_b).wait()
        process(t + 1, in_b)
        return 0

    lax.fori_loop(0, N_TILES // 2, pair_body, 0)
    pltpu.sync_copy(
        outv_v.at[pl.ds(0, RPW * TOPK)],
        vals_hbm.at[pl.ds(row0 * TOPK, RPW * TOPK)],
    )
    pltpu.sync_copy(
        outi_v.at[pl.ds(0, RPW * TOPK)],
        idx_hbm.at[pl.ds(row0 * TOPK, RPW * TOPK)],
    )


def _topk(coeff_flat):
    mesh = plsc.VectorSubcoreMesh(
        core_axis_name="c", subcore_axis_name="s", num_cores=NC, num_subcores=NS
    )
    return pl.kernel(
        _topk_body,
        out_type=(
            jax.ShapeDtypeStruct((ROWS * TOPK,), jnp.float32),
            jax.ShapeDtypeStruct((ROWS * TOPK,), jnp.int32),
        ),
        mesh=mesh,
        compiler_params=pltpu.CompilerParams(needs_layout_passes=False),
        scratch_types=[
            pltpu.VMEM((R_TILE * POOL,), jnp.float32),
            pltpu.VMEM((R_TILE * POOL,), jnp.float32),
            pltpu.VMEM((RPW * TOPK + L,), jnp.float32),
            pltpu.VMEM((RPW * TOPK + L,), jnp.int32),
            pltpu.SemaphoreType.DMA,
            pltpu.SemaphoreType.DMA,
        ],
    )(coeff_flat)


def kernel(query_in, W1, b1, W2, b2, W3, b3):
    coeff = _mlp(query_in, W1, b1, W2, b2, W3, b3)
    vals_flat, idx_flat = _topk(coeff.reshape(-1))
    topk_coeff = vals_flat.reshape(N_TOKENS, GROUPS * TOPK)
    topk_idx = idx_flat.reshape(N_TOKENS, GROUPS * TOPK)
    return coeff, topk_coeff, topk_idx
